# baseline (speedup 1.0000x reference)
import functools

import jax
import jax.numpy as jnp
from jax import lax
from jax.experimental import pallas as pl
from jax.experimental.pallas import tpu as pltpu
from jax.experimental.pallas import tpu_sc as plsc

_NUM_CORES = 2
_NUM_SUBCORES = 16
_LANES = 16
_NW = _NUM_CORES * _NUM_SUBCORES


def _build_sc_kernel(B, V, D):
    assert B % _NW == 0
    b_per_w = B // _NW
    CH = 64
    NSLOT = 6
    assert b_per_w % CH == 0
    n_chunks = b_per_w // CH
    assert D % 8 == 0

    mesh = plsc.VectorSubcoreMesh(core_axis_name="c", subcore_axis_name="s")

    @functools.partial(
        pl.kernel,
        out_type=jax.ShapeDtypeStruct((B,), jnp.float32),
        mesh=mesh,
        compiler_params=pltpu.CompilerParams(needs_layout_passes=False),
        scratch_types=[
            pltpu.VMEM((n_chunks, CH), jnp.int32),
            pltpu.VMEM((n_chunks, CH), jnp.int32),
            pltpu.VMEM((NSLOT * CH, D), jnp.float32),
            pltpu.VMEM((NSLOT * CH, D), jnp.float32),
            pltpu.VMEM((b_per_w,), jnp.float32),
            pltpu.SemaphoreType.DMA,
            pltpu.SemaphoreType.DMA,
            pltpu.SemaphoreType.DMA,
            pltpu.SemaphoreType.DMA,
            pltpu.SemaphoreType.DMA,
            pltpu.SemaphoreType.DMA,
            pltpu.SemaphoreType.DMA,
        ],
    )
    def sc_kernel(cw_hbm, xw_hbm, win_hbm, wout_hbm, out_hbm,
                  idx_c, idx_x, cbuf, xbuf, out_v,
                  sem0, sem1, sem2, sem3, sem4, sem5, semi):
        wid = lax.axis_index("s") * _NUM_CORES + lax.axis_index("c")
        base = wid * b_per_w

        idx_waits = []
        for ch in range(n_chunks):
            idx_waits.append((
                pltpu.async_copy(
                    cw_hbm.at[pl.ds(base + ch * CH, CH)], idx_c.at[ch], semi),
                pltpu.async_copy(
                    xw_hbm.at[pl.ds(base + ch * CH, CH)], idx_x.at[ch], semi)))

        sems = (sem0, sem1, sem2, sem3, sem4, sem5)

        def start(ch):
            slot = ch % NSLOT
            idx_waits[ch][0].wait()
            idx_waits[ch][1].wait()
            d1 = pltpu.async_copy(win_hbm.at[idx_c.at[ch]],
                                  cbuf.at[pl.ds(slot * CH, CH)], sems[slot])
            d2 = pltpu.async_copy(wout_hbm.at[idx_x.at[ch]],
                                  xbuf.at[pl.ds(slot * CH, CH)], sems[slot])
            return d1, d2

        lane = lax.broadcasted_iota(jnp.int32, (_LANES,), 0)
        inflight = [start(ch) for ch in range(NSLOT - 1)]
        for ch in range(n_chunks):
            if ch + NSLOT - 1 < n_chunks:
                inflight.append(start(ch + NSLOT - 1))
            pending = inflight.pop(0)
            pending[0].wait()
            pending[1].wait()
            slot = ch % NSLOT
            for g in range(CH // _LANES):
                row_v = lane + (slot * CH + g * _LANES)

                def dim_body(i, carry, row_v=row_v):
                    accs, col_v = carry
                    accs = list(accs)
                    for j in range(16):
                        c = plsc.load_gather(cbuf, [row_v, col_v])
                        x = plsc.load_gather(xbuf, [row_v, col_v])
                        accs[j % 4] = accs[j % 4] + c * x
                        col_v = (col_v + 1) & (D - 1)
                    return tuple(accs), col_v

                zero = jnp.zeros((_LANES,), jnp.float32)
                (a0, a1, a2, a3), _ = lax.fori_loop(
                    0, D // 16, dim_body,
                    ((zero, zero, zero, zero), lane))
                acc = (a0 + a1) + (a2 + a3)
                out_v[pl.ds(ch * CH + g * _LANES, _LANES)] = acc

        pltpu.sync_copy(out_v, out_hbm.at[pl.ds(base, b_per_w)])

    return sc_kernel


def kernel(center_word, context_words, W_in, W_out):
    B = center_word.shape[0]
    V, D = W_in.shape
    cw = center_word.astype(jnp.int32)
    xw = context_words.astype(jnp.int32)
    sc = _build_sc_kernel(B, V, D)
    return sc(cw, xw, W_in, W_out)


# speedup vs baseline: 1.0891x; 1.0891x over previous
import functools

import jax
import jax.numpy as jnp
from jax import lax
from jax.experimental import pallas as pl
from jax.experimental.pallas import tpu as pltpu
from jax.experimental.pallas import tpu_sc as plsc

_NUM_CORES = 2
_NUM_SUBCORES = 16
_LANES = 16
_NW = _NUM_CORES * _NUM_SUBCORES


def _build_sc_kernel(B, V, D):
    assert B % _NW == 0
    b_per_w = B // _NW
    CH = 64
    NSLOT = 6
    assert b_per_w % CH == 0
    n_chunks = b_per_w // CH
    assert D % 8 == 0

    mesh = plsc.VectorSubcoreMesh(core_axis_name="c", subcore_axis_name="s")

    @functools.partial(
        pl.kernel,
        out_type=jax.ShapeDtypeStruct((B,), jnp.float32),
        mesh=mesh,
        compiler_params=pltpu.CompilerParams(needs_layout_passes=False),
        scratch_types=[
            pltpu.VMEM((n_chunks, CH), jnp.int32),
            pltpu.VMEM((n_chunks, CH), jnp.int32),
            pltpu.VMEM((NSLOT * CH, D), jnp.float32),
            pltpu.VMEM((NSLOT * CH, D), jnp.float32),
            pltpu.VMEM((b_per_w,), jnp.float32),
            pltpu.SemaphoreType.DMA,
            pltpu.SemaphoreType.DMA,
            pltpu.SemaphoreType.DMA,
            pltpu.SemaphoreType.DMA,
            pltpu.SemaphoreType.DMA,
            pltpu.SemaphoreType.DMA,
            pltpu.SemaphoreType.DMA,
        ],
    )
    def sc_kernel(cw_hbm, xw_hbm, win_hbm, wout_hbm, out_hbm,
                  idx_c, idx_x, cbuf, xbuf, out_v,
                  sem0, sem1, sem2, sem3, sem4, sem5, semi):
        wid = lax.axis_index("s") * _NUM_CORES + lax.axis_index("c")
        base = wid * b_per_w

        idx_waits = []
        for ch in range(n_chunks):
            idx_waits.append((
                pltpu.async_copy(
                    cw_hbm.at[pl.ds(base + ch * CH, CH)], idx_c.at[ch], semi),
                pltpu.async_copy(
                    xw_hbm.at[pl.ds(base + ch * CH, CH)], idx_x.at[ch], semi)))

        sems = (sem0, sem1, sem2, sem3, sem4, sem5)

        def start(ch):
            slot = ch % NSLOT
            idx_waits[ch][0].wait()
            idx_waits[ch][1].wait()
            d1 = pltpu.async_copy(win_hbm.at[idx_c.at[ch]],
                                  cbuf.at[pl.ds(slot * CH, CH)], sems[slot],
                                  priority=1)
            d2 = pltpu.async_copy(wout_hbm.at[idx_x.at[ch]],
                                  xbuf.at[pl.ds(slot * CH, CH)], sems[slot],
                                  priority=1)
            return d1, d2

        lane = lax.broadcasted_iota(jnp.int32, (_LANES,), 0)
        inflight = [start(ch) for ch in range(NSLOT - 1)]
        for ch in range(n_chunks):
            if ch + NSLOT - 1 < n_chunks:
                inflight.append(start(ch + NSLOT - 1))
            pending = inflight.pop(0)
            pending[0].wait()
            pending[1].wait()
            slot = ch % NSLOT
            for g in range(CH // _LANES):
                row_v = lane + (slot * CH + g * _LANES)

                def dim_body(i, carry, row_v=row_v):
                    acc, col_v = carry
                    for _ in range(8):
                        c = plsc.load_gather(cbuf, [row_v, col_v])
                        x = plsc.load_gather(xbuf, [row_v, col_v])
                        acc = acc + c * x
                        col_v = (col_v + 1) & (D - 1)
                    return acc, col_v

                acc, _ = lax.fori_loop(
                    0, D // 8, dim_body,
                    (jnp.zeros((_LANES,), jnp.float32), lane))
                out_v[pl.ds(ch * CH + g * _LANES, _LANES)] = acc

        pltpu.sync_copy(out_v, out_hbm.at[pl.ds(base, b_per_w)])

    return sc_kernel


def kernel(center_word, context_words, W_in, W_out):
    B = center_word.shape[0]
    V, D = W_in.shape
    cw = center_word.astype(jnp.int32)
    xw = context_words.astype(jnp.int32)
    sc = _build_sc_kernel(B, V, D)
    return sc(cw, xw, W_in, W_out)


# speedup vs baseline: 1.0892x; 1.0000x over previous
import functools

import jax
import jax.numpy as jnp
from jax import lax
from jax.experimental import pallas as pl
from jax.experimental.pallas import tpu as pltpu
from jax.experimental.pallas import tpu_sc as plsc

_NUM_CORES = 2
_NUM_SUBCORES = 16
_LANES = 16
_NW = _NUM_CORES * _NUM_SUBCORES


def _build_sc_kernel(B, V, D):
    assert B % _NW == 0
    b_per_w = B // _NW
    CH = 64
    NSLOT = 7
    assert b_per_w % CH == 0
    n_chunks = b_per_w // CH
    assert D % 8 == 0

    mesh = plsc.VectorSubcoreMesh(core_axis_name="c", subcore_axis_name="s")

    @functools.partial(
        pl.kernel,
        out_type=jax.ShapeDtypeStruct((B,), jnp.float32),
        mesh=mesh,
        compiler_params=pltpu.CompilerParams(needs_layout_passes=False),
        scratch_types=[
            pltpu.VMEM((n_chunks, CH), jnp.int32),
            pltpu.VMEM((n_chunks, CH), jnp.int32),
            pltpu.VMEM((NSLOT * CH, D), jnp.float32),
            pltpu.VMEM((NSLOT * CH, D), jnp.float32),
            pltpu.VMEM((b_per_w,), jnp.float32),
            pltpu.SemaphoreType.DMA,
            pltpu.SemaphoreType.DMA,
            pltpu.SemaphoreType.DMA,
            pltpu.SemaphoreType.DMA,
            pltpu.SemaphoreType.DMA,
            pltpu.SemaphoreType.DMA,
            pltpu.SemaphoreType.DMA,
            pltpu.SemaphoreType.DMA,
        ],
    )
    def sc_kernel(cw_hbm, xw_hbm, win_hbm, wout_hbm, out_hbm,
                  idx_c, idx_x, cbuf, xbuf, out_v,
                  sem0, sem1, sem2, sem3, sem4, sem5, sem6, semi):
        wid = lax.axis_index("s") * _NUM_CORES + lax.axis_index("c")
        base = wid * b_per_w

        idx_waits = []
        for ch in range(n_chunks):
            idx_waits.append((
                pltpu.async_copy(
                    cw_hbm.at[pl.ds(base + ch * CH, CH)], idx_c.at[ch], semi),
                pltpu.async_copy(
                    xw_hbm.at[pl.ds(base + ch * CH, CH)], idx_x.at[ch], semi)))

        sems = (sem0, sem1, sem2, sem3, sem4, sem5, sem6)

        def start(ch):
            slot = ch % NSLOT
            idx_waits[ch][0].wait()
            idx_waits[ch][1].wait()
            d1 = pltpu.async_copy(win_hbm.at[idx_c.at[ch]],
                                  cbuf.at[pl.ds(slot * CH, CH)], sems[slot],
                                  priority=1)
            d2 = pltpu.async_copy(wout_hbm.at[idx_x.at[ch]],
                                  xbuf.at[pl.ds(slot * CH, CH)], sems[slot],
                                  priority=1)
            return d1, d2

        lane = lax.broadcasted_iota(jnp.int32, (_LANES,), 0)
        inflight = [start(ch) for ch in range(NSLOT - 1)]
        for ch in range(n_chunks):
            if ch + NSLOT - 1 < n_chunks:
                inflight.append(start(ch + NSLOT - 1))
            pending = inflight.pop(0)
            pending[0].wait()
            pending[1].wait()
            slot = ch % NSLOT
            for g in range(CH // _LANES):
                row_v = lane + (slot * CH + g * _LANES)

                def dim_body(i, carry, row_v=row_v):
                    acc, col_v = carry
                    for _ in range(8):
                        c = plsc.load_gather(cbuf, [row_v, col_v])
                        x = plsc.load_gather(xbuf, [row_v, col_v])
                        acc = acc + c * x
                        col_v = (col_v + 1) & (D - 1)
                    return acc, col_v

                acc, _ = lax.fori_loop(
                    0, D // 8, dim_body,
                    (jnp.zeros((_LANES,), jnp.float32), lane))
                out_v[pl.ds(ch * CH + g * _LANES, _LANES)] = acc

        pltpu.sync_copy(out_v, out_hbm.at[pl.ds(base, b_per_w)])

    return sc_kernel


def kernel(center_word, context_words, W_in, W_out):
    B = center_word.shape[0]
    V, D = W_in.shape
    cw = center_word.astype(jnp.int32)
    xw = context_words.astype(jnp.int32)
    sc = _build_sc_kernel(B, V, D)
    return sc(cw, xw, W_in, W_out)


# speedup vs baseline: 1.1582x; 1.0633x over previous
import functools

import jax
import jax.numpy as jnp
from jax import lax
from jax.experimental import pallas as pl
from jax.experimental.pallas import tpu as pltpu
from jax.experimental.pallas import tpu_sc as plsc

_NUM_CORES = 2
_NUM_SUBCORES = 16
_LANES = 16
_NW = _NUM_CORES * _NUM_SUBCORES


def _build_sc_kernel(B, V, D):
    assert B % _NW == 0
    b_per_w = B // _NW
    CH = 64
    NSLOT = 7
    assert b_per_w % CH == 0
    n_chunks = b_per_w // CH
    assert D % 8 == 0

    mesh = plsc.VectorSubcoreMesh(core_axis_name="c", subcore_axis_name="s")

    @functools.partial(
        pl.kernel,
        out_type=jax.ShapeDtypeStruct((B,), jnp.float32),
        mesh=mesh,
        compiler_params=pltpu.CompilerParams(needs_layout_passes=False),
        scratch_types=[
            pltpu.VMEM((n_chunks, CH), jnp.int32),
            pltpu.VMEM((n_chunks, CH), jnp.int32),
            pltpu.VMEM((NSLOT * CH, D), jnp.float32),
            pltpu.VMEM((NSLOT * CH, D), jnp.float32),
            pltpu.VMEM((b_per_w,), jnp.float32),
            pltpu.SemaphoreType.DMA,
            pltpu.SemaphoreType.DMA,
            pltpu.SemaphoreType.DMA,
            pltpu.SemaphoreType.DMA,
            pltpu.SemaphoreType.DMA,
            pltpu.SemaphoreType.DMA,
            pltpu.SemaphoreType.DMA,
            pltpu.SemaphoreType.DMA,
        ],
    )
    def sc_kernel(cw_hbm, xw_hbm, win_hbm, wout_hbm, out_hbm,
                  idx_c, idx_x, cbuf, xbuf, out_v,
                  sem0, sem1, sem2, sem3, sem4, sem5, sem6, semi):
        wid = lax.axis_index("s") * _NUM_CORES + lax.axis_index("c")
        base = wid * b_per_w

        idx_waits = []
        for ch in range(n_chunks):
            idx_waits.append((
                pltpu.async_copy(
                    cw_hbm.at[pl.ds(base + ch * CH, CH)], idx_c.at[ch], semi),
                pltpu.async_copy(
                    xw_hbm.at[pl.ds(base + ch * CH, CH)], idx_x.at[ch], semi)))

        sems = (sem0, sem1, sem2, sem3, sem4, sem5, sem6)

        def start(ch):
            slot = ch % NSLOT
            idx_waits[ch][0].wait()
            idx_waits[ch][1].wait()
            d1 = pltpu.async_copy(win_hbm.at[idx_c.at[ch]],
                                  cbuf.at[pl.ds(slot * CH, CH)], sems[slot],
                                  priority=1)
            d2 = pltpu.async_copy(wout_hbm.at[idx_x.at[ch]],
                                  xbuf.at[pl.ds(slot * CH, CH)], sems[slot],
                                  priority=1)
            return d1, d2

        lane = lax.broadcasted_iota(jnp.int32, (_LANES,), 0)
        lastmask = lane == (_LANES - 1)
        inflight = [start(ch) for ch in range(NSLOT - 1)]
        for ch in range(n_chunks):
            if ch + NSLOT - 1 < n_chunks:
                inflight.append(start(ch + NSLOT - 1))
            pending = inflight.pop(0)
            pending[0].wait()
            pending[1].wait()
            slot = ch % NSLOT

            @plsc.parallel_loop(0, CH, step=1, unroll=2)
            def elem_loop(e, ch=ch, slot=slot):
                crow = slot * CH + e
                a = cbuf[crow, pl.ds(0, _LANES)] * xbuf[crow, pl.ds(0, _LANES)]
                b = (cbuf[crow, pl.ds(_LANES, _LANES)]
                     * xbuf[crow, pl.ds(_LANES, _LANES)])
                for k in range(2, D // _LANES, 2):
                    a = a + (cbuf[crow, pl.ds(k * _LANES, _LANES)]
                             * xbuf[crow, pl.ds(k * _LANES, _LANES)])
                    b = b + (cbuf[crow, pl.ds((k + 1) * _LANES, _LANES)]
                             * xbuf[crow, pl.ds((k + 1) * _LANES, _LANES)])
                s = plsc.cumsum(a + b)
                pos = jnp.full((_LANES,), ch * CH + e, jnp.int32)
                plsc.store_scatter(out_v, [pos], s, mask=lastmask)

        pltpu.sync_copy(out_v, out_hbm.at[pl.ds(base, b_per_w)])

    return sc_kernel


def kernel(center_word, context_words, W_in, W_out):
    B = center_word.shape[0]
    V, D = W_in.shape
    cw = center_word.astype(jnp.int32)
    xw = context_words.astype(jnp.int32)
    sc = _build_sc_kernel(B, V, D)
    return sc(cw, xw, W_in, W_out)
